# Initial kernel scaffold; baseline (speedup 1.0000x reference)
import functools

import jax
import jax.numpy as jnp
from jax import lax
from jax.experimental import pallas as pl
from jax.experimental.pallas import tpu as pltpu
from jax.experimental.pallas import tpu_sc as plsc

N = 10000
D = 128
E = 320000
NW = 32
WPB = 320
NPAD = NW * WPB
C = 4000
NV = C // 16
NCHUNK = E // C
K = 112
KB = K + 16
NEG = -3.0e38


def _sc_agg(feat, src, dst):
    mesh = plsc.VectorSubcoreMesh(core_axis_name="c", subcore_axis_name="s")

    @functools.partial(
        pl.kernel,
        out_type=jax.ShapeDtypeStruct((NPAD, D), jnp.float32),
        mesh=mesh,
        scratch_types=[
            pltpu.VMEM((C,), jnp.int32),
            pltpu.VMEM((C,), jnp.int32),
            pltpu.VMEM((KB,), jnp.int32),
            pltpu.VMEM((KB,), jnp.int32),
            pltpu.VMEM((KB, D), jnp.float32),
            pltpu.VMEM((WPB, D), jnp.float32),
            pltpu.SemaphoreType.DMA,
        ],
    )
    def agg_kernel(feat_hbm, src_hbm, dst_hbm, out_hbm,
                   dstb, srcb, ssrc, sdst, rows, acc, gsem):
        wid = lax.axis_index("s") * 2 + lax.axis_index("c")
        lo = wid * WPB

        zero16 = jnp.zeros((16,), jnp.int32)
        neg16 = jnp.full((16,), NEG, jnp.float32)

        def init_sel(i, _):
            ssrc[pl.ds(i * 16, 16)] = zero16
            return 0
        lax.fori_loop(0, KB // 16, init_sel, 0)

        def init_acc(r, _):
            for j in range(D // 16):
                acc[r, pl.ds(j * 16, 16)] = neg16
            return 0
        lax.fori_loop(0, WPB, init_acc, 0)

        def consume(nproc):
            pltpu.async_copy(feat_hbm.at[ssrc], rows, gsem).wait()

            def acc_one(k, _):
                dloc = sdst[k]
                for j in range(D // 16):
                    sl = pl.ds(j * 16, 16)
                    acc[dloc, sl] = jnp.maximum(acc[dloc, sl], rows[k, sl])
                return 0
            lax.fori_loop(0, nproc, acc_one, 0)

        def flush_full(cur):
            consume(K)
            s = ssrc[pl.ds(K, 16)]
            d = sdst[pl.ds(K, 16)]
            ssrc[pl.ds(0, 16)] = s
            sdst[pl.ds(0, 16)] = d
            return cur - K

        def chunk_body(c, cursor):
            pltpu.sync_copy(dst_hbm.at[pl.ds(c * C, C)], dstb)
            pltpu.sync_copy(src_hbm.at[pl.ds(c * C, C)], srcb)

            def vec_body(i, cur):
                sl = pl.ds(i * 16, 16)
                dloc = dstb[sl] - lo
                sv = srcb[sl]
                m = (dloc >= 0) & (dloc < WPB)
                mi = jnp.where(m, 1, 0).astype(jnp.int32)
                incl = plsc.cumsum(mi)
                pos = (incl - mi) + cur
                plsc.store_scatter(ssrc, [pos], sv, mask=m)
                plsc.store_scatter(sdst, [pos], dloc, mask=m)
                cur = cur + jnp.sum(mi)
                cur = lax.cond(cur >= K, flush_full, lambda v: v, cur)
                return cur
            return lax.fori_loop(0, NV, vec_body, cursor)

        cursor = lax.fori_loop(0, NCHUNK, chunk_body, 0)
        consume(cursor)

        thresh = jnp.full((16,), -1.0e38, jnp.float32)
        zf = jnp.zeros((16,), jnp.float32)

        def fixup(r, _):
            for j in range(D // 16):
                sl = pl.ds(j * 16, 16)
                v = acc[r, sl]
                acc[r, sl] = jnp.where(v > thresh, v, zf)
            return 0
        lax.fori_loop(0, WPB, fixup, 0)

        pltpu.sync_copy(acc, out_hbm.at[pl.ds(lo, WPB)])

    return agg_kernel(feat, src, dst)


def _tc_linear(agg, x, w_l, w_r, b, relu):
    BR = 1000

    def body(agg_ref, x_ref, wl_ref, wr_ref, b_ref, o_ref):
        r = jnp.dot(agg_ref[...], wl_ref[...],
                    preferred_element_type=jnp.float32)
        r = r + jnp.dot(x_ref[...], wr_ref[...],
                        preferred_element_type=jnp.float32)
        r = r + b_ref[...]
        if relu:
            r = jnp.maximum(r, 0.0)
        o_ref[...] = r

    return pl.pallas_call(
        body,
        grid=(N // BR,),
        in_specs=[
            pl.BlockSpec((BR, D), lambda i: (i, 0)),
            pl.BlockSpec((BR, D), lambda i: (i, 0)),
            pl.BlockSpec((D, D), lambda i: (0, 0)),
            pl.BlockSpec((D, D), lambda i: (0, 0)),
            pl.BlockSpec((1, D), lambda i: (0, 0)),
        ],
        out_specs=pl.BlockSpec((BR, D), lambda i: (i, 0)),
        out_shape=jax.ShapeDtypeStruct((N, D), jnp.float32),
    )(agg, x, w_l, w_r, b.reshape(1, D))


def kernel(x, edge_index, W1_l, W1_r, b1, W2_l, W2_r, b2):
    src = edge_index[0]
    dst = edge_index[1]
    agg1 = _sc_agg(x, src, dst)
    h = _tc_linear(agg1, x, W1_l, W1_r, b1, relu=True)
    agg2 = _sc_agg(h, src, dst)
    out = _tc_linear(agg2, h, W2_l, W2_r, b2, relu=False)
    return out


# speedup vs baseline: 1.4562x; 1.4562x over previous
import functools

import jax
import jax.numpy as jnp
from jax import lax
from jax.experimental import pallas as pl
from jax.experimental.pallas import tpu as pltpu
from jax.experimental.pallas import tpu_sc as plsc

N = 10000
D = 128
E = 320000
NW = 32
WPB = 320
NPAD = NW * WPB
C = 4000
NV = C // 16
NCHUNK = E // C
K = 112
KB = K + 16
NEG = -3.0e38
BR = 1000


def _sc_agg(feat, src, dst):
    mesh = plsc.VectorSubcoreMesh(core_axis_name="c", subcore_axis_name="s")

    @functools.partial(
        pl.kernel,
        out_type=jax.ShapeDtypeStruct((NPAD, D), jnp.float32),
        mesh=mesh,
        compiler_params=pltpu.CompilerParams(needs_layout_passes=False),
        scratch_types=[
            pltpu.VMEM((C,), jnp.int32),
            pltpu.VMEM((C,), jnp.int32),
            pltpu.VMEM((KB,), jnp.int32),
            pltpu.VMEM((KB + 16,), jnp.int32),
            pltpu.VMEM((KB, D), jnp.float32),
            pltpu.VMEM((WPB, D), jnp.float32),
            pltpu.SemaphoreType.DMA,
        ],
    )
    def agg_kernel(feat_hbm, src_hbm, dst_hbm, out_hbm,
                   dstb, srcb, ssrc, sdst, rows, acc, gsem):
        wid = lax.axis_index("s") * 2 + lax.axis_index("c")
        lo = wid * WPB

        zero16 = jnp.zeros((16,), jnp.int32)
        neg16 = jnp.full((16,), NEG, jnp.float32)

        def init_sel(i, _):
            ssrc[pl.ds(i * 16, 16)] = zero16
            return 0
        lax.fori_loop(0, KB // 16, init_sel, 0)

        def init_acc(r, _):
            for j in range(D // 16):
                acc[r, pl.ds(j * 16, 16)] = neg16
            return 0
        lax.fori_loop(0, WPB, init_acc, 0)

        def consume(nproc):
            pltpu.async_copy(feat_hbm.at[ssrc], rows, gsem).wait()

            def acc_one(k, _):
                dloc = sdst[pl.ds(k, 16)][0]
                for j in range(D // 16):
                    sl = pl.ds(j * 16, 16)
                    acc[dloc, sl] = jnp.maximum(acc[dloc, sl], rows[k, sl])
                return 0
            lax.fori_loop(0, nproc, acc_one, 0)

        def flush_all(cur):
            consume(cur)
            return jnp.int32(0)

        def chunk_body(c, cursor):
            pltpu.sync_copy(dst_hbm.at[pl.ds(c * C, C)], dstb)
            pltpu.sync_copy(src_hbm.at[pl.ds(c * C, C)], srcb)

            def vec_body(i, cur):
                cur = lax.cond(cur >= K, flush_all, lambda v: v, cur)
                sl = pl.ds(i * 16, 16)
                dloc = dstb[sl] - lo
                sv = srcb[sl]
                m = (dloc >= 0) & (dloc < WPB)
                iota = lax.iota(jnp.int32, 16)
                key = jnp.where(m, iota, iota + 16)
                _, sv_s = plsc.sort_key_val(key, sv)
                _, dl_s = plsc.sort_key_val(key, dloc)
                ssrc[pl.ds(cur, 16)] = sv_s
                sdst[pl.ds(cur, 16)] = dl_s
                cur = cur + plsc.all_reduce_population_count(m)[0]
                return cur
            return lax.fori_loop(0, NV, vec_body, cursor)

        cursor = lax.fori_loop(0, NCHUNK, chunk_body, 0)
        consume(cursor)

        thresh = jnp.full((16,), -1.0e38, jnp.float32)
        zf = jnp.zeros((16,), jnp.float32)

        def fixup(r, _):
            for j in range(D // 16):
                sl = pl.ds(j * 16, 16)
                v = acc[r, sl]
                acc[r, sl] = jnp.where(v > thresh, v, zf)
            return 0
        lax.fori_loop(0, WPB, fixup, 0)

        pltpu.sync_copy(acc, out_hbm.at[pl.ds(lo, WPB)])

    return agg_kernel(feat, src, dst)


def _tc_linear(agg, x, w_l, w_r, b, relu):
    def body(agg_ref, x_ref, wl_ref, wr_ref, b_ref, o_ref):
        r = jnp.dot(agg_ref[...], wl_ref[...],
                    preferred_element_type=jnp.float32)
        r = r + jnp.dot(x_ref[...], wr_ref[...],
                        preferred_element_type=jnp.float32)
        r = r + b_ref[...]
        if relu:
            r = jnp.maximum(r, 0.0)
        o_ref[...] = r

    return pl.pallas_call(
        body,
        grid=(N // BR,),
        in_specs=[
            pl.BlockSpec((BR, D), lambda i: (i, 0)),
            pl.BlockSpec((BR, D), lambda i: (i, 0)),
            pl.BlockSpec((D, D), lambda i: (0, 0)),
            pl.BlockSpec((D, D), lambda i: (0, 0)),
            pl.BlockSpec((1, D), lambda i: (0, 0)),
        ],
        out_specs=pl.BlockSpec((BR, D), lambda i: (i, 0)),
        out_shape=jax.ShapeDtypeStruct((N, D), jnp.float32),
    )(agg, x, w_l, w_r, b.reshape(1, D))


def kernel(x, edge_index, W1_l, W1_r, b1, W2_l, W2_r, b2):
    src = edge_index[0]
    dst = edge_index[1]
    agg1 = _sc_agg(x, src, dst)
    h = _tc_linear(agg1, x, W1_l, W1_r, b1, relu=True)
    agg2 = _sc_agg(h, src, dst)
    out = _tc_linear(agg2, h, W2_l, W2_r, b2, relu=False)
    return out


# speedup vs baseline: 1.8938x; 1.3005x over previous
import functools

import jax
import jax.numpy as jnp
from jax import lax
from jax.experimental import pallas as pl
from jax.experimental.pallas import tpu as pltpu
from jax.experimental.pallas import tpu_sc as plsc

N = 10000
D = 128
E = 320000
NW = 32
WPB = 320
NPAD = NW * WPB
C = 4000
NV = C // 16
NCHUNK = E // C
K = 112
KB = K + 16
NEG = -3.0e38
BR = 1000


def _sc_agg(feat, src, dst):
    mesh = plsc.VectorSubcoreMesh(core_axis_name="c", subcore_axis_name="s")

    @functools.partial(
        pl.kernel,
        out_type=jax.ShapeDtypeStruct((NPAD, D), jnp.float32),
        mesh=mesh,
        compiler_params=pltpu.CompilerParams(needs_layout_passes=False),
        scratch_types=[
            pltpu.VMEM((2 * C,), jnp.int32),
            pltpu.VMEM((2 * C,), jnp.int32),
            pltpu.VMEM((2, KB), jnp.int32),
            pltpu.VMEM((2, KB + 16), jnp.int32),
            pltpu.VMEM((2, KB, D), jnp.float32),
            pltpu.VMEM((WPB, D), jnp.float32),
            pltpu.SemaphoreType.DMA((2,)),
            pltpu.SemaphoreType.DMA((2,)),
        ],
    )
    def agg_kernel(feat_hbm, src_hbm, dst_hbm, out_hbm,
                   dstb, srcb, ssrc, sdst, rows, acc, gsem, esem):
        wid = lax.axis_index("s") * 2 + lax.axis_index("c")
        lo = wid * WPB

        zero16 = jnp.zeros((16,), jnp.int32)
        neg16 = jnp.full((16,), NEG, jnp.float32)

        def init_sel(i, _):
            ssrc[0, pl.ds(i * 16, 16)] = zero16
            ssrc[1, pl.ds(i * 16, 16)] = zero16
            return 0
        lax.fori_loop(0, KB // 16, init_sel, 0)

        def init_acc(r, _):
            for j in range(D // 16):
                acc[r, pl.ds(j * 16, 16)] = neg16
            return 0
        lax.fori_loop(0, WPB, init_acc, 0)

        def acc_batch(bp):
            def grp(g, _):
                dvec = sdst[bp, pl.ds(g * 16, 16)]
                for t in range(16):
                    dloc = dvec[t]
                    for j in range(D // 16):
                        sl = pl.ds(j * 16, 16)
                        acc[dloc, sl] = jnp.maximum(
                            acc[dloc, sl], rows[bp, g * 16 + t, sl])
                return 0
            lax.fori_loop(0, K // 16, grp, 0)

        def acc_tail(bp, nproc):
            def acc_one(k, _):
                dloc = sdst[bp, pl.ds(k, 16)][0]
                for j in range(D // 16):
                    sl = pl.ds(j * 16, 16)
                    acc[dloc, sl] = jnp.maximum(
                        acc[dloc, sl], rows[bp, k, sl])
                return 0
            lax.fori_loop(0, nproc, acc_one, 0)

        def wait_gather(bp):
            pltpu.make_async_copy(
                feat_hbm.at[ssrc.at[bp]], rows.at[bp], gsem.at[bp]).wait()

        def do_flush(args):
            cur, par, pending = args
            npar = 1 - par
            pltpu.async_copy(
                feat_hbm.at[ssrc.at[par]], rows.at[par], gsem.at[par])

            def do_acc(_):
                wait_gather(npar)
                acc_batch(npar)
                return 0
            lax.cond(pending == 1, do_acc, lambda u: 0, 0)

            ssrc[npar, pl.ds(0, 16)] = ssrc[par, pl.ds(K, 16)]
            sdst[npar, pl.ds(0, 16)] = sdst[par, pl.ds(K, 16)]
            return (cur - K, npar, jnp.int32(1))

        def fire_chunk(c, p):
            pltpu.async_copy(dst_hbm.at[pl.ds(c * C, C)],
                             dstb.at[pl.ds(p * C, C)], esem.at[p])
            pltpu.async_copy(src_hbm.at[pl.ds(c * C, C)],
                             srcb.at[pl.ds(p * C, C)], esem.at[p])

        fire_chunk(0, 0)

        def chunk_body(c, carry):
            p = jnp.bitwise_and(c, 1)
            pltpu.make_async_copy(dst_hbm.at[pl.ds(c * C, C)],
                                  dstb.at[pl.ds(p * C, C)],
                                  esem.at[p]).wait()
            pltpu.make_async_copy(src_hbm.at[pl.ds(c * C, C)],
                                  srcb.at[pl.ds(p * C, C)],
                                  esem.at[p]).wait()

            def prefetch(_):
                fire_chunk(c + 1, 1 - p)
                return 0
            lax.cond(c + 1 < NCHUNK, prefetch, lambda u: 0, 0)

            def vec_body(i, st):
                cur, par, pending = st
                st = lax.cond(cur >= K, do_flush, lambda v: v,
                              (cur, par, pending))
                cur, par, pending = st
                dloc = dstb[pl.ds(p * C + i * 16, 16)] - lo
                sv = srcb[pl.ds(p * C + i * 16, 16)]
                m = (dloc >= 0) & (dloc < WPB)
                iota = lax.iota(jnp.int32, 16)
                key = jnp.where(m, iota, iota + 16)
                _, sv_s = plsc.sort_key_val(key, sv)
                _, dl_s = plsc.sort_key_val(key, dloc)
                ssrc[par, pl.ds(cur, 16)] = sv_s
                sdst[par, pl.ds(cur, 16)] = dl_s
                cur = cur + plsc.all_reduce_population_count(m)[0]
                return (cur, par, pending)
            return lax.fori_loop(0, NV, vec_body, carry, unroll=5)

        cur, par, pending = lax.fori_loop(
            0, NCHUNK, chunk_body, (jnp.int32(0), jnp.int32(0), jnp.int32(0)))

        def final_acc(_):
            wait_gather(1 - par)
            acc_batch(1 - par)
            return 0
        lax.cond(pending == 1, final_acc, lambda u: 0, 0)
        pltpu.async_copy(feat_hbm.at[ssrc.at[par]], rows.at[par],
                         gsem.at[par])
        wait_gather(par)
        acc_tail(par, cur)

        thresh = jnp.full((16,), -1.0e38, jnp.float32)
        zf = jnp.zeros((16,), jnp.float32)

        def fixup(r, _):
            for j in range(D // 16):
                sl = pl.ds(j * 16, 16)
                v = acc[r, sl]
                acc[r, sl] = jnp.where(v > thresh, v, zf)
            return 0
        lax.fori_loop(0, WPB, fixup, 0)

        pltpu.sync_copy(acc, out_hbm.at[pl.ds(lo, WPB)])

    return agg_kernel(feat, src, dst)


def _tc_linear(agg, x, w_l, w_r, b, relu):
    def body(agg_ref, x_ref, wl_ref, wr_ref, b_ref, o_ref):
        r = jnp.dot(agg_ref[...], wl_ref[...],
                    preferred_element_type=jnp.float32)
        r = r + jnp.dot(x_ref[...], wr_ref[...],
                        preferred_element_type=jnp.float32)
        r = r + b_ref[...]
        if relu:
            r = jnp.maximum(r, 0.0)
        o_ref[...] = r

    return pl.pallas_call(
        body,
        grid=(N // BR,),
        in_specs=[
            pl.BlockSpec((BR, D), lambda i: (i, 0)),
            pl.BlockSpec((BR, D), lambda i: (i, 0)),
            pl.BlockSpec((D, D), lambda i: (0, 0)),
            pl.BlockSpec((D, D), lambda i: (0, 0)),
            pl.BlockSpec((1, D), lambda i: (0, 0)),
        ],
        out_specs=pl.BlockSpec((BR, D), lambda i: (i, 0)),
        out_shape=jax.ShapeDtypeStruct((N, D), jnp.float32),
    )(agg, x, w_l, w_r, b.reshape(1, D))


def kernel(x, edge_index, W1_l, W1_r, b1, W2_l, W2_r, b2):
    src = edge_index[0]
    dst = edge_index[1]
    agg1 = _sc_agg(x, src, dst)
    h = _tc_linear(agg1, x, W1_l, W1_r, b1, relu=True)
    agg2 = _sc_agg(h, src, dst)
    out = _tc_linear(agg2, h, W2_l, W2_r, b2, relu=False)
    return out
